# Initial kernel scaffold; baseline (speedup 1.0000x reference)
import functools

import jax
import jax.numpy as jnp
import numpy as np
from jax import lax
from jax.experimental import pallas as pl
from jax.experimental.pallas import tpu as pltpu
from jax.experimental.pallas import tpu_sc as plsc

D = 32
NC = 2
NS = 16
NW = NC * NS
C = 1024
SUB = 128
SCALE = np.float32(np.sqrt(np.float32(32.0)))


@functools.lru_cache(maxsize=None)
def _make_kernel(B: int):
  T = B // NW
  G = T // C
  assert T % C == 0 and C % SUB == 0

  mesh = plsc.VectorSubcoreMesh(core_axis_name="c", subcore_axis_name="s")

  @functools.partial(
      pl.kernel,
      out_type=jax.ShapeDtypeStruct((B, D), jnp.float32),
      mesh=mesh,
      scratch_types=[
          pltpu.VMEM((C,), jnp.int32),
          pltpu.VMEM((C, D), jnp.float32),
          pltpu.SemaphoreType.DMA,
      ],
  )
  def emb_kernel(tokens_hbm, table_hbm, out_hbm, idx_v, rows_v, sem):
    wid = lax.axis_index("s") * NC + lax.axis_index("c")
    base = wid * T

    @pl.loop(0, G)
    def chunk_loop(g):
      off = base + g * C
      pltpu.sync_copy(tokens_hbm.at[pl.ds(off, C)], idx_v)

      copies = [
          pltpu.async_copy(
              table_hbm.at[idx_v.at[pl.ds(j * SUB, SUB)]],
              rows_v.at[pl.ds(j * SUB, SUB)],
              sem,
          )
          for j in range(C // SUB)
      ]
      for cp in copies:
        cp.wait()

      @pl.loop(0, C, unroll=8)
      def scale_loop(r):
        for h in range(2):
          sl = rows_v[r, pl.ds(h * 16, 16)]
          rows_v[r, pl.ds(h * 16, 16)] = sl * SCALE

      pltpu.sync_copy(rows_v, out_hbm.at[pl.ds(off, C)])

  return emb_kernel


@jax.jit
def kernel(tokens, table):
  B = tokens.shape[0] * tokens.shape[1]
  flat = tokens.reshape(B)
  out = _make_kernel(B)(flat, table)
  return out.reshape(tokens.shape + (D,))


# speedup vs baseline: 1.3999x; 1.3999x over previous
import functools

import jax
import jax.numpy as jnp
import numpy as np
from jax import lax
from jax.experimental import pallas as pl
from jax.experimental.pallas import tpu as pltpu
from jax.experimental.pallas import tpu_sc as plsc

D = 32
NC = 2
NS = 16
NW = NC * NS
C = 1024
SUB = 128
SCALE = np.float32(np.sqrt(np.float32(32.0)))


@functools.lru_cache(maxsize=None)
def _make_kernel(B: int):
  T = B // NW
  G = T // C
  assert T % C == 0 and C % SUB == 0

  mesh = plsc.VectorSubcoreMesh(core_axis_name="c", subcore_axis_name="s")

  @functools.partial(
      pl.kernel,
      out_type=jax.ShapeDtypeStruct((B, D), jnp.float32),
      mesh=mesh,
      scratch_types=[
          pltpu.VMEM((C,), jnp.int32),
          pltpu.VMEM((C, D), jnp.float32),
          pltpu.SemaphoreType.DMA,
      ],
      compiler_params=pltpu.CompilerParams(use_tc_tiling_on_sc=False),
  )
  def emb_kernel(tokens_hbm, table_hbm, out_hbm, idx_v, rows_v, sem):
    wid = lax.axis_index("s") * NC + lax.axis_index("c")
    base = wid * T

    @pl.loop(0, G)
    def chunk_loop(g):
      off = base + g * C
      pltpu.sync_copy(tokens_hbm.at[pl.ds(off, C)], idx_v)

      copies = [
          pltpu.async_copy(
              table_hbm.at[idx_v.at[pl.ds(j * SUB, SUB)]],
              rows_v.at[pl.ds(j * SUB, SUB)],
              sem,
          )
          for j in range(C // SUB)
      ]
      for cp in copies:
        cp.wait()

      @pl.loop(0, C, unroll=8)
      def scale_loop(r):
        for h in range(2):
          sl = rows_v[r, pl.ds(h * 16, 16)]
          rows_v[r, pl.ds(h * 16, 16)] = sl * SCALE

      pltpu.sync_copy(rows_v, out_hbm.at[pl.ds(off, C)])

  return emb_kernel


@jax.jit
def kernel(tokens, table):
  B = tokens.shape[0] * tokens.shape[1]
  flat = tokens.reshape(B)
  out = _make_kernel(B)(flat, table)
  return out.reshape(tokens.shape + (D,))
